# baseline (speedup 1.0000x reference)
import jax
import jax.numpy as jnp
from jax.experimental import pallas as pl
from jax.experimental.pallas import tpu as pltpu

N = 10000
NFEAT = 128
NHID = 16

BM = 256


def _xw_kernel(x_ref, w_ref, h_ref):
    h_ref[...] = jnp.dot(x_ref[...], w_ref[...],
                         preferred_element_type=jnp.float32)


def _gcn_kernel(adj_ref, x_ref, w_ref, b_ref, eps_ref,
                z_ref, mu_ref, std_ref, h_ref):
    @pl.when(pl.program_id(0) == 0)
    def _compute_h():
        h_ref[...] = jnp.dot(x_ref[...], w_ref[...],
                             preferred_element_type=jnp.float32)

    acc_t = jax.lax.dot_general(
        h_ref[...], adj_ref[...],
        dimension_numbers=(((0,), (1,)), ((), ())),
        preferred_element_type=jnp.float32)
    r = jnp.maximum(acc_t + b_ref[...], 0.0)
    mu = r[:NHID, :]
    std = jnp.exp(0.5 * r[NHID:, :])
    mu_ref[...] = mu
    std_ref[...] = std
    z_ref[...] = mu + std * eps_ref[...].T


def kernel(x, adj, W1, b1, W2, b2, eps):
    Wcat = jnp.concatenate([W1, W2], axis=1)
    bcat = jnp.concatenate([b1, b2]).reshape(2 * NHID, 1)

    z_t, mu_t, std_t = pl.pallas_call(
        _gcn_kernel,
        grid=(pl.cdiv(N, BM),),
        in_specs=[
            pl.BlockSpec((BM, N), lambda m: (m, 0)),
            pl.BlockSpec((N, NFEAT), lambda m: (0, 0)),
            pl.BlockSpec((NFEAT, 2 * NHID), lambda m: (0, 0)),
            pl.BlockSpec((2 * NHID, 1), lambda m: (0, 0)),
            pl.BlockSpec((BM, NHID), lambda m: (m, 0)),
        ],
        out_specs=[
            pl.BlockSpec((NHID, BM), lambda m: (0, m)),
            pl.BlockSpec((NHID, BM), lambda m: (0, m)),
            pl.BlockSpec((NHID, BM), lambda m: (0, m)),
        ],
        out_shape=[
            jax.ShapeDtypeStruct((NHID, N), jnp.float32),
            jax.ShapeDtypeStruct((NHID, N), jnp.float32),
            jax.ShapeDtypeStruct((NHID, N), jnp.float32),
        ],
        scratch_shapes=[pltpu.VMEM((N, 2 * NHID), jnp.float32)],
        compiler_params=pltpu.CompilerParams(
            dimension_semantics=("arbitrary",),
        ),
    )(adj, x, Wcat, bcat, eps)
    return (z_t.T, mu_t.T, std_t.T)


# speedup vs baseline: 1.0428x; 1.0428x over previous
import jax
import jax.numpy as jnp
from jax.experimental import pallas as pl
from jax.experimental.pallas import tpu as pltpu

N = 10000
NFEAT = 128
NHID = 16

BM = 256


def _gcn_kernel(adj_ref, x_ref, w_ref, b_ref, eps_ref,
                z_ref, mu_ref, std_ref, h_ref):
    @pl.when(pl.program_id(0) == 0)
    def _compute_h():
        h_ref[...] = jnp.dot(x_ref[...], w_ref[...],
                             preferred_element_type=jnp.float32)

    acc_t = jax.lax.dot_general(
        h_ref[...], adj_ref[...],
        dimension_numbers=(((0,), (1,)), ((), ())),
        preferred_element_type=jnp.float32)
    r = jnp.maximum(acc_t + b_ref[...], 0.0)
    mu = r[:NHID, :]
    std = jnp.exp(0.5 * r[NHID:, :])
    mu_ref[...] = mu
    std_ref[...] = std
    z_ref[...] = mu + std * eps_ref[...]


def kernel(x, adj, W1, b1, W2, b2, eps):
    Wcat = jnp.concatenate([W1, W2], axis=1)
    bcat = jnp.concatenate([b1, b2]).reshape(2 * NHID, 1)
    eps_t = eps.T

    z_t, mu_t, std_t = pl.pallas_call(
        _gcn_kernel,
        grid=(pl.cdiv(N, BM),),
        in_specs=[
            pl.BlockSpec((BM, N), lambda m: (m, 0)),
            pl.BlockSpec((N, NFEAT), lambda m: (0, 0)),
            pl.BlockSpec((NFEAT, 2 * NHID), lambda m: (0, 0)),
            pl.BlockSpec((2 * NHID, 1), lambda m: (0, 0)),
            pl.BlockSpec((NHID, BM), lambda m: (0, m)),
        ],
        out_specs=[
            pl.BlockSpec((NHID, BM), lambda m: (0, m)),
            pl.BlockSpec((NHID, BM), lambda m: (0, m)),
            pl.BlockSpec((NHID, BM), lambda m: (0, m)),
        ],
        out_shape=[
            jax.ShapeDtypeStruct((NHID, N), jnp.float32),
            jax.ShapeDtypeStruct((NHID, N), jnp.float32),
            jax.ShapeDtypeStruct((NHID, N), jnp.float32),
        ],
        scratch_shapes=[pltpu.VMEM((N, 2 * NHID), jnp.float32)],
        compiler_params=pltpu.CompilerParams(
            dimension_semantics=("arbitrary",),
        ),
    )(adj, x, Wcat, bcat, eps_t)
    return (z_t.T, mu_t.T, std_t.T)
